# Initial kernel scaffold; baseline (speedup 1.0000x reference)
import jax
import jax.numpy as jnp
from jax.experimental import pallas as pl


def kernel(x, coords, edge_index, edge_attr, W1e, b1e, g1e, be1e, W2e, b2e, Wse, bse, W1n, b1n, g1n, be1n, W2n, b2n, gnn, bnn):
    raise NotImplementedError("write your pallas kernel here")


# speedup vs baseline: 4.4848x; 4.4848x over previous
import functools

import jax
import jax.numpy as jnp
from jax import lax
from jax.experimental import pallas as pl
from jax.experimental.pallas import tpu as pltpu
from jax.experimental.pallas import tpu_sc as plsc

N = 10000
E = 320000
D = 128
A = 16
MW = 144
NPAD = 10240

NC = 2
NS = 16
NW = NC * NS
EPW = E // NW
C = 80
NCHUNK = EPW // C

_SIGMAS = (0.1, 0.5, 1.0)


def _prep_body(x_ref, ws_ref, wd_ref, b_ref, tp_ref, tq_ref):
    xb = x_ref[...]
    tp_ref[...] = (
        jnp.dot(xb, ws_ref[...], preferred_element_type=jnp.float32) + b_ref[...]
    )
    tq_ref[...] = jnp.dot(xb, wd_ref[...], preferred_element_type=jnp.float32)


def _prep(x, w_src, w_dst, b1e):
    bn = 1000
    return pl.pallas_call(
        _prep_body,
        grid=(N // bn,),
        in_specs=[
            pl.BlockSpec((bn, D), lambda i: (i, 0)),
            pl.BlockSpec((D, D), lambda i: (0, 0)),
            pl.BlockSpec((D, D), lambda i: (0, 0)),
            pl.BlockSpec((1, D), lambda i: (0, 0)),
        ],
        out_specs=[
            pl.BlockSpec((bn, D), lambda i: (i, 0)),
            pl.BlockSpec((bn, D), lambda i: (i, 0)),
        ],
        out_shape=[
            jax.ShapeDtypeStruct((N, D), jnp.float32),
            jax.ShapeDtypeStruct((N, D), jnp.float32),
        ],
    )(x, w_src, w_dst, b1e)


@functools.lru_cache(maxsize=None)
def _make_mesh():
    return plsc.VectorSubcoreMesh(
        core_axis_name="c", subcore_axis_name="s",
        num_cores=NC, num_subcores=NS)


@functools.lru_cache(maxsize=None)
def _make_sc_gather():
    return functools.partial(
        pl.kernel,
        out_type=(
            jax.ShapeDtypeStruct((E, D), jnp.float32),
            jax.ShapeDtypeStruct((E,), jnp.float32),
        ),
        mesh=_make_mesh(),
        compiler_params=pltpu.CompilerParams(needs_layout_passes=False),
        scratch_types=[
            pltpu.VMEM((N,), jnp.float32),
            pltpu.VMEM((N,), jnp.float32),
            pltpu.VMEM((N,), jnp.float32),
            pltpu.VMEM((C,), jnp.int32),
            pltpu.VMEM((C,), jnp.int32),
            pltpu.VMEM((C, D), jnp.float32),
            pltpu.VMEM((C, D), jnp.float32),
            pltpu.VMEM((C,), jnp.float32),
            pltpu.SemaphoreType.DMA,
            pltpu.SemaphoreType.DMA,
        ],
    )(_sc_gather_body)


def _sc_gather_body(tp_hbm, tq_hbm, cx_hbm, cy_hbm, cz_hbm, src_hbm, dst_hbm,
                    sum_out, rd_out,
                    cx_v, cy_v, cz_v, src_v, dst_v, bufp, bufq, rd_v,
                    semp, semq):
    wid = lax.axis_index("s") * NC + lax.axis_index("c")
    pltpu.sync_copy(cx_hbm, cx_v)
    pltpu.sync_copy(cy_hbm, cy_v)
    pltpu.sync_copy(cz_hbm, cz_v)

    def chunk(c, _):
        base = wid * EPW + c * C
        pltpu.sync_copy(src_hbm.at[pl.ds(base, C)], src_v)
        pltpu.sync_copy(dst_hbm.at[pl.ds(base, C)], dst_v)
        cp1 = pltpu.async_copy(tp_hbm.at[src_v], bufp, semp)
        cp2 = pltpu.async_copy(tq_hbm.at[dst_v], bufq, semq)
        cp1.wait()
        cp2.wait()
        for g in range(C // 16):
            s16 = src_v[pl.ds(g * 16, 16)]
            d16 = dst_v[pl.ds(g * 16, 16)]
            acc = jnp.zeros((16,), jnp.float32)
            for cv in (cx_v, cy_v, cz_v):
                r = plsc.load_gather(cv, [s16]) - plsc.load_gather(cv, [d16])
                acc = acc + r * r
            rd_v[pl.ds(g * 16, 16)] = acc

        def row(i, _):
            for k in range(D // 16):
                sl = pl.ds(k * 16, 16)
                plsc.addupdate(bufp.at[i, sl], bufq[i, sl])
            return 0

        lax.fori_loop(0, C, row, 0, unroll=False)
        pltpu.sync_copy(bufp, sum_out.at[pl.ds(base, C)])
        pltpu.sync_copy(rd_v, rd_out.at[pl.ds(base, C)])
        return 0

    lax.fori_loop(0, NCHUNK, chunk, 0, unroll=False)


def _edge_body(sum_ref, rd_ref, ea_ref, wea_ref, wrbf_ref, g1_ref, be1_ref,
               w2_ref, b2_ref, wse_ref, bse_ref, out_ref):
    b2 = sum_ref.shape[0]
    h1 = sum_ref[...] + jnp.dot(
        ea_ref[...], wea_ref[...], preferred_element_type=jnp.float32)
    d2 = rd_ref[...] + 1e-8
    wr = wrbf_ref[...]
    for k, sig in enumerate(_SIGMAS):
        h1 = h1 + jnp.exp(d2 * (-1.0 / (2.0 * sig * sig))) * wr[k:k + 1, :]
    h = h1 * jax.nn.sigmoid(h1)
    mu = jnp.mean(h, axis=-1, keepdims=True)
    var = jnp.mean((h - mu) ** 2, axis=-1, keepdims=True)
    h = (h - mu) * lax.rsqrt(var + 1e-5) * g1_ref[...] + be1_ref[...]
    m = jnp.dot(h, w2_ref[...], preferred_element_type=jnp.float32) + b2_ref[...]
    w = jax.nn.sigmoid(
        jnp.dot(m, wse_ref[...], preferred_element_type=jnp.float32) + bse_ref[...])
    m = m * w
    out_ref[...] = m


def _edge(summ, rd, ea, wea, wrbf, g1, be1, w2, b2, wse, bse):
    blk = 2000
    return pl.pallas_call(
        _edge_body,
        grid=(E // blk,),
        in_specs=[
            pl.BlockSpec((blk, D), lambda i: (i, 0)),
            pl.BlockSpec((blk, 1), lambda i: (i, 0)),
            pl.BlockSpec((blk, A), lambda i: (i, 0)),
            pl.BlockSpec((A, D), lambda i: (0, 0)),
            pl.BlockSpec((3, D), lambda i: (0, 0)),
            pl.BlockSpec((1, D), lambda i: (0, 0)),
            pl.BlockSpec((1, D), lambda i: (0, 0)),
            pl.BlockSpec((D, D), lambda i: (0, 0)),
            pl.BlockSpec((1, D), lambda i: (0, 0)),
            pl.BlockSpec((D, 1), lambda i: (0, 0)),
            pl.BlockSpec((1, 1), lambda i: (0, 0)),
        ],
        out_specs=pl.BlockSpec((blk, D), lambda i: (i, 0)),
        out_shape=jax.ShapeDtypeStruct((E, D), jnp.float32),
    )(summ, rd, ea, wea, wrbf, g1, be1, w2, b2, wse, bse)


@functools.lru_cache(maxsize=None)
def _make_sc_scatter():
    return functools.partial(
        pl.kernel,
        out_type=(
            jax.ShapeDtypeStruct((NC, NPAD, D), jnp.float32),
            jax.ShapeDtypeStruct((NW, NPAD), jnp.int32),
        ),
        mesh=_make_mesh(),
        compiler_params=pltpu.CompilerParams(needs_layout_passes=False),
        scratch_types=[
            pltpu.VMEM((C, D), jnp.float32),
            pltpu.VMEM((C,), jnp.int32),
            pltpu.VMEM((NPAD,), jnp.int32),
            pltpu.VMEM_SHARED((NPAD, D), jnp.float32),
        ],
    )(_sc_scatter_body)


def _sc_scatter_body(m_hbm, dst_hbm, out_hbm, cnt_hbm, m_v, dsti_v, cnt_v,
                     acc_sh):
    cid = lax.axis_index("c")
    sid = lax.axis_index("s")
    wid = sid * NC + cid
    rows_per_tile = NPAD // NS

    def zrow(i, _):
        for k in range(D // 16):
            m_v[i, pl.ds(k * 16, 16)] = jnp.zeros((16,), jnp.float32)
        return 0

    lax.fori_loop(0, C, zrow, 0, unroll=False)

    def zcnt(i, _):
        cnt_v[pl.ds(i * 16, 16)] = jnp.zeros((16,), jnp.int32)
        return 0

    lax.fori_loop(0, NPAD // 16, zcnt, 0, unroll=False)

    def zcp(b, _):
        pltpu.sync_copy(m_v, acc_sh.at[pl.ds(sid * rows_per_tile + b * C, C)])
        return 0

    lax.fori_loop(0, rows_per_tile // C, zcp, 0, unroll=False)
    plsc.subcore_barrier()

    def chunk(c, _):
        base = wid * EPW + c * C
        pltpu.sync_copy(dst_hbm.at[pl.ds(base, C)], dsti_v)
        pltpu.sync_copy(m_hbm.at[pl.ds(base, C)], m_v)
        pltpu.sync_copy(m_v, acc_sh.at[dsti_v], add=True)
        for g in range(C // 16):
            v16 = dsti_v[pl.ds(g * 16, 16)]
            cnts, last = plsc.scan_count(v16)
            cur = plsc.load_gather(cnt_v, [v16])
            plsc.store_scatter(cnt_v, [v16], cur + cnts, mask=last)
        return 0

    lax.fori_loop(0, NCHUNK, chunk, 0, unroll=False)
    pltpu.sync_copy(cnt_v, cnt_hbm.at[wid])
    plsc.subcore_barrier()

    def ocp(b, _):
        r0 = sid * rows_per_tile + b * C
        pltpu.sync_copy(acc_sh.at[pl.ds(r0, C)], m_v)
        pltpu.sync_copy(m_v, out_hbm.at[cid].at[pl.ds(r0, C)])
        return 0

    lax.fori_loop(0, rows_per_tile // C, ocp, 0, unroll=False)


def _node_body(x_ref, parts_ref, cnt_ref, w1x_ref, w1a_ref, b1_ref, g1_ref,
               be1_ref, w2_ref, b2_ref, gn_ref, bn_ref, out_ref):
    xb = x_ref[...]
    p = parts_ref[0] + parts_ref[1]
    cnt = jnp.sum(cnt_ref[...], axis=0).astype(jnp.float32)[:, None]
    agg = p / jnp.maximum(cnt, 1.0)
    h = (jnp.dot(xb, w1x_ref[...], preferred_element_type=jnp.float32)
         + jnp.dot(agg, w1a_ref[...], preferred_element_type=jnp.float32)
         + b1_ref[...])
    h = h * jax.nn.sigmoid(h)
    mu = jnp.mean(h, axis=-1, keepdims=True)
    var = jnp.mean((h - mu) ** 2, axis=-1, keepdims=True)
    h = (h - mu) * lax.rsqrt(var + 1e-5) * g1_ref[...] + be1_ref[...]
    h = jnp.dot(h, w2_ref[...], preferred_element_type=jnp.float32) + b2_ref[...]
    mu = jnp.mean(h, axis=-1, keepdims=True)
    var = jnp.mean((h - mu) ** 2, axis=-1, keepdims=True)
    h = (h - mu) * lax.rsqrt(var + 1e-5) * gn_ref[...] + bn_ref[...]
    out_ref[...] = xb + h


def _node(x, parts, cnth, w1x, w1a, b1, g1, be1, w2, b2, gn, bn):
    bn_blk = 1024
    return pl.pallas_call(
        _node_body,
        grid=(NPAD // bn_blk,),
        in_specs=[
            pl.BlockSpec((bn_blk, D), lambda i: (i, 0)),
            pl.BlockSpec((NC, bn_blk, D), lambda i: (0, i, 0)),
            pl.BlockSpec((NW, bn_blk), lambda i: (0, i)),
            pl.BlockSpec((D, D), lambda i: (0, 0)),
            pl.BlockSpec((D, D), lambda i: (0, 0)),
            pl.BlockSpec((1, D), lambda i: (0, 0)),
            pl.BlockSpec((1, D), lambda i: (0, 0)),
            pl.BlockSpec((1, D), lambda i: (0, 0)),
            pl.BlockSpec((D, D), lambda i: (0, 0)),
            pl.BlockSpec((1, D), lambda i: (0, 0)),
            pl.BlockSpec((1, D), lambda i: (0, 0)),
            pl.BlockSpec((1, D), lambda i: (0, 0)),
        ],
        out_specs=pl.BlockSpec((bn_blk, D), lambda i: (i, 0)),
        out_shape=jax.ShapeDtypeStruct((NPAD, D), jnp.float32),
    )(x, parts, cnth, w1x, w1a, b1, g1, be1, w2, b2, gn, bn)


def kernel(x, coords, edge_index, edge_attr,
           W1e, b1e, g1e, be1e, W2e, b2e, Wse, bse,
           W1n, b1n, g1n, be1n, W2n, b2n, gnn, bnn):
    src = edge_index[0].astype(jnp.int32)
    dst = edge_index[1].astype(jnp.int32)
    w_src = W1e[:D]
    w_dst = W1e[D:2 * D]
    w_ea = W1e[2 * D:2 * D + A]
    w_rbf = W1e[2 * D + A:]

    tp, tq = _prep(x, w_src, w_dst, b1e.reshape(1, D))
    summ, rd = _make_sc_gather()(tp, tq, coords[:, 0], coords[:, 1],
                                 coords[:, 2], src, dst)
    m = _edge(summ, rd.reshape(E, 1), edge_attr,
              w_ea, w_rbf, g1e.reshape(1, D), be1e.reshape(1, D),
              W2e, b2e.reshape(1, D), Wse, bse.reshape(1, 1))
    parts, cnth = _make_sc_scatter()(m, dst)
    x_pad = jnp.pad(x, ((0, NPAD - N), (0, 0)))
    out = _node(x_pad, parts, cnth, W1n[:D], W1n[D:], b1n.reshape(1, D),
                g1n.reshape(1, D), be1n.reshape(1, D),
                W2n, b2n.reshape(1, D), gnn.reshape(1, D), bnn.reshape(1, D))
    return out[:N]


# speedup vs baseline: 5.7907x; 1.2912x over previous
import functools

import jax
import jax.numpy as jnp
from jax import lax
from jax.experimental import pallas as pl
from jax.experimental.pallas import tpu as pltpu
from jax.experimental.pallas import tpu_sc as plsc

N = 10000
E = 320000
D = 128
A = 16
MW = 144
NPAD = 10240

NC = 2
NS = 16
NW = NC * NS
EPW = E // NW
C = 80
NCHUNK = EPW // C

_SIGMAS = (0.1, 0.5, 1.0)


def _prep_body(x_ref, ws_ref, wd_ref, b_ref, tp_ref, tq_ref):
    xb = x_ref[...]
    tp_ref[...] = (
        jnp.dot(xb, ws_ref[...], preferred_element_type=jnp.float32) + b_ref[...]
    )
    tq_ref[...] = jnp.dot(xb, wd_ref[...], preferred_element_type=jnp.float32)


def _prep(x, w_src, w_dst, b1e):
    bn = 1000
    return pl.pallas_call(
        _prep_body,
        grid=(N // bn,),
        in_specs=[
            pl.BlockSpec((bn, D), lambda i: (i, 0)),
            pl.BlockSpec((D, D), lambda i: (0, 0)),
            pl.BlockSpec((D, D), lambda i: (0, 0)),
            pl.BlockSpec((1, D), lambda i: (0, 0)),
        ],
        out_specs=[
            pl.BlockSpec((bn, D), lambda i: (i, 0)),
            pl.BlockSpec((bn, D), lambda i: (i, 0)),
        ],
        out_shape=[
            jax.ShapeDtypeStruct((N, D), jnp.float32),
            jax.ShapeDtypeStruct((N, D), jnp.float32),
        ],
    )(x, w_src, w_dst, b1e)


@functools.lru_cache(maxsize=None)
def _make_mesh():
    return plsc.VectorSubcoreMesh(
        core_axis_name="c", subcore_axis_name="s",
        num_cores=NC, num_subcores=NS)


@functools.lru_cache(maxsize=None)
def _make_sc_gather():
    return functools.partial(
        pl.kernel,
        out_type=(
            jax.ShapeDtypeStruct((E, D), jnp.float32),
            jax.ShapeDtypeStruct((E,), jnp.float32),
        ),
        mesh=_make_mesh(),
        compiler_params=pltpu.CompilerParams(needs_layout_passes=False),
        scratch_types=[
            pltpu.VMEM((N,), jnp.float32),
            pltpu.VMEM((N,), jnp.float32),
            pltpu.VMEM((N,), jnp.float32),
            pltpu.VMEM((C,), jnp.int32),
            pltpu.VMEM((C,), jnp.int32),
            pltpu.VMEM((C, D), jnp.float32),
            pltpu.VMEM((C, D), jnp.float32),
            pltpu.VMEM((C,), jnp.float32),
            pltpu.VMEM((C,), jnp.int32),
            pltpu.VMEM((C,), jnp.int32),
            pltpu.VMEM((C, D), jnp.float32),
            pltpu.VMEM((C, D), jnp.float32),
            pltpu.VMEM((C,), jnp.float32),
            pltpu.SemaphoreType.DMA,
            pltpu.SemaphoreType.DMA,
        ],
    )(_sc_gather_body)


def _sc_gather_body(tp_hbm, tq_hbm, cx_hbm, cy_hbm, cz_hbm, src_hbm, dst_hbm,
                    sum_out, rd_out,
                    cx_v, cy_v, cz_v,
                    src0, dst0, bufp0, bufq0, rd0,
                    src1, dst1, bufp1, bufq1, rd1,
                    sem0, sem1):
    wid = lax.axis_index("s") * NC + lax.axis_index("c")
    pltpu.sync_copy(cx_hbm, cx_v)
    pltpu.sync_copy(cy_hbm, cy_v)
    pltpu.sync_copy(cz_hbm, cz_v)
    sets = ((src0, dst0, bufp0, bufq0, rd0, sem0),
            (src1, dst1, bufp1, bufq1, rd1, sem1))

    def stage_in(c, s):
        src_v, dst_v, bufp, bufq, _, sem = sets[s]
        base = wid * EPW + c * C
        pltpu.sync_copy(src_hbm.at[pl.ds(base, C)], src_v)
        pltpu.sync_copy(dst_hbm.at[pl.ds(base, C)], dst_v)
        pltpu.async_copy(tp_hbm.at[src_v], bufp, sem)
        pltpu.async_copy(tq_hbm.at[dst_v], bufq, sem)

    def process(c, s):
        src_v, dst_v, bufp, bufq, rd_v, sem = sets[s]
        base = wid * EPW + c * C
        for g in range(C // 16):
            s16 = src_v[pl.ds(g * 16, 16)]
            d16 = dst_v[pl.ds(g * 16, 16)]
            acc = jnp.zeros((16,), jnp.float32)
            for cv in (cx_v, cy_v, cz_v):
                r = plsc.load_gather(cv, [s16]) - plsc.load_gather(cv, [d16])
                acc = acc + r * r
            rd_v[pl.ds(g * 16, 16)] = acc
        pltpu.make_async_copy(tp_hbm.at[src_v], bufp, sem).wait()
        pltpu.make_async_copy(tq_hbm.at[dst_v], bufq, sem).wait()

        def row(i, _):
            for k in range(D // 16):
                sl = pl.ds(k * 16, 16)
                plsc.addupdate(bufp.at[i, sl], bufq[i, sl])
            return 0

        lax.fori_loop(0, C, row, 0, unroll=False)
        pltpu.sync_copy(bufp, sum_out.at[pl.ds(base, C)])
        pltpu.sync_copy(rd_v, rd_out.at[pl.ds(base, C)])

    stage_in(0, 0)

    def group(g, _):
        c0 = 2 * g
        stage_in(c0 + 1, 1)
        process(c0, 0)
        stage_in(c0 + 2, 0)
        process(c0 + 1, 1)
        return 0

    lax.fori_loop(0, (NCHUNK - 1) // 2, group, 0, unroll=False)
    process(NCHUNK - 1, 0)


def _edge_body(sum_ref, rd_ref, ea_ref, wea_ref, wrbf_ref, g1_ref, be1_ref,
               w2_ref, b2_ref, wse_ref, bse_ref, out_ref):
    b2 = sum_ref.shape[0]
    h1 = sum_ref[...] + jnp.dot(
        ea_ref[...], wea_ref[...], preferred_element_type=jnp.float32)
    d2 = rd_ref[...] + 1e-8
    wr = wrbf_ref[...]
    for k, sig in enumerate(_SIGMAS):
        h1 = h1 + jnp.exp(d2 * (-1.0 / (2.0 * sig * sig))) * wr[k:k + 1, :]
    h = h1 * jax.nn.sigmoid(h1)
    mu = jnp.mean(h, axis=-1, keepdims=True)
    var = jnp.mean((h - mu) ** 2, axis=-1, keepdims=True)
    h = (h - mu) * lax.rsqrt(var + 1e-5) * g1_ref[...] + be1_ref[...]
    m = jnp.dot(h, w2_ref[...], preferred_element_type=jnp.float32) + b2_ref[...]
    w = jax.nn.sigmoid(
        jnp.dot(m, wse_ref[...], preferred_element_type=jnp.float32) + bse_ref[...])
    m = m * w
    out_ref[...] = m


def _edge(summ, rd, ea, wea, wrbf, g1, be1, w2, b2, wse, bse):
    blk = 2000
    return pl.pallas_call(
        _edge_body,
        grid=(E // blk,),
        in_specs=[
            pl.BlockSpec((blk, D), lambda i: (i, 0)),
            pl.BlockSpec((blk, 1), lambda i: (i, 0)),
            pl.BlockSpec((blk, A), lambda i: (i, 0)),
            pl.BlockSpec((A, D), lambda i: (0, 0)),
            pl.BlockSpec((3, D), lambda i: (0, 0)),
            pl.BlockSpec((1, D), lambda i: (0, 0)),
            pl.BlockSpec((1, D), lambda i: (0, 0)),
            pl.BlockSpec((D, D), lambda i: (0, 0)),
            pl.BlockSpec((1, D), lambda i: (0, 0)),
            pl.BlockSpec((D, 1), lambda i: (0, 0)),
            pl.BlockSpec((1, 1), lambda i: (0, 0)),
        ],
        out_specs=pl.BlockSpec((blk, D), lambda i: (i, 0)),
        out_shape=jax.ShapeDtypeStruct((E, D), jnp.float32),
    )(summ, rd, ea, wea, wrbf, g1, be1, w2, b2, wse, bse)


@functools.lru_cache(maxsize=None)
def _make_sc_scatter():
    return functools.partial(
        pl.kernel,
        out_type=(
            jax.ShapeDtypeStruct((NC, NPAD, D), jnp.float32),
            jax.ShapeDtypeStruct((NW, NPAD), jnp.int32),
        ),
        mesh=_make_mesh(),
        compiler_params=pltpu.CompilerParams(needs_layout_passes=False),
        scratch_types=[
            pltpu.VMEM((C, D), jnp.float32),
            pltpu.VMEM((C,), jnp.int32),
            pltpu.VMEM((C, D), jnp.float32),
            pltpu.VMEM((C,), jnp.int32),
            pltpu.VMEM((NPAD,), jnp.int32),
            pltpu.VMEM_SHARED((NPAD, D), jnp.float32),
            pltpu.SemaphoreType.DMA,
            pltpu.SemaphoreType.DMA,
        ],
    )(_sc_scatter_body)


def _sc_scatter_body(m_hbm, dst_hbm, out_hbm, cnt_hbm,
                     m0, di0, m1, di1, cnt_v, acc_sh, sem0, sem1):
    cid = lax.axis_index("c")
    sid = lax.axis_index("s")
    wid = sid * NC + cid
    rows_per_tile = NPAD // NS
    sets = ((m0, di0, sem0), (m1, di1, sem1))

    def zrow(i, _):
        for k in range(D // 16):
            m0[i, pl.ds(k * 16, 16)] = jnp.zeros((16,), jnp.float32)
        return 0

    lax.fori_loop(0, C, zrow, 0, unroll=False)

    def zcnt(i, _):
        cnt_v[pl.ds(i * 16, 16)] = jnp.zeros((16,), jnp.int32)
        return 0

    lax.fori_loop(0, NPAD // 16, zcnt, 0, unroll=False)

    def zcp(b, _):
        pltpu.sync_copy(m0, acc_sh.at[pl.ds(sid * rows_per_tile + b * C, C)])
        return 0

    lax.fori_loop(0, rows_per_tile // C, zcp, 0, unroll=False)
    plsc.subcore_barrier()

    def stage_in(c, s):
        m_v, dsti_v, sem = sets[s]
        base = wid * EPW + c * C
        pltpu.sync_copy(dst_hbm.at[pl.ds(base, C)], dsti_v)
        pltpu.async_copy(m_hbm.at[pl.ds(base, C)], m_v, sem)

    def process(c, s):
        m_v, dsti_v, sem = sets[s]
        base = wid * EPW + c * C
        for g in range(C // 16):
            v16 = dsti_v[pl.ds(g * 16, 16)]
            cnts, last = plsc.scan_count(v16)
            cur = plsc.load_gather(cnt_v, [v16])
            plsc.store_scatter(cnt_v, [v16], cur + cnts, mask=last)
        pltpu.make_async_copy(m_hbm.at[pl.ds(base, C)], m_v, sem).wait()
        pltpu.sync_copy(m_v, acc_sh.at[dsti_v], add=True)

    stage_in(0, 0)

    def group(g, _):
        c0 = 2 * g
        stage_in(c0 + 1, 1)
        process(c0, 0)
        stage_in(c0 + 2, 0)
        process(c0 + 1, 1)
        return 0

    lax.fori_loop(0, (NCHUNK - 1) // 2, group, 0, unroll=False)
    process(NCHUNK - 1, 0)
    pltpu.sync_copy(cnt_v, cnt_hbm.at[wid])
    plsc.subcore_barrier()

    def ocp(b, _):
        r0 = sid * rows_per_tile + b * C
        pltpu.sync_copy(acc_sh.at[pl.ds(r0, C)], m0)
        pltpu.sync_copy(m0, out_hbm.at[cid].at[pl.ds(r0, C)])
        return 0

    lax.fori_loop(0, rows_per_tile // C, ocp, 0, unroll=False)


def _node_body(x_ref, parts_ref, cnt_ref, w1x_ref, w1a_ref, b1_ref, g1_ref,
               be1_ref, w2_ref, b2_ref, gn_ref, bn_ref, out_ref):
    xb = x_ref[...]
    p = parts_ref[0] + parts_ref[1]
    cnt = jnp.sum(cnt_ref[...], axis=0).astype(jnp.float32)[:, None]
    agg = p / jnp.maximum(cnt, 1.0)
    h = (jnp.dot(xb, w1x_ref[...], preferred_element_type=jnp.float32)
         + jnp.dot(agg, w1a_ref[...], preferred_element_type=jnp.float32)
         + b1_ref[...])
    h = h * jax.nn.sigmoid(h)
    mu = jnp.mean(h, axis=-1, keepdims=True)
    var = jnp.mean((h - mu) ** 2, axis=-1, keepdims=True)
    h = (h - mu) * lax.rsqrt(var + 1e-5) * g1_ref[...] + be1_ref[...]
    h = jnp.dot(h, w2_ref[...], preferred_element_type=jnp.float32) + b2_ref[...]
    mu = jnp.mean(h, axis=-1, keepdims=True)
    var = jnp.mean((h - mu) ** 2, axis=-1, keepdims=True)
    h = (h - mu) * lax.rsqrt(var + 1e-5) * gn_ref[...] + bn_ref[...]
    out_ref[...] = xb + h


def _node(x, parts, cnth, w1x, w1a, b1, g1, be1, w2, b2, gn, bn):
    bn_blk = 1024
    return pl.pallas_call(
        _node_body,
        grid=(NPAD // bn_blk,),
        in_specs=[
            pl.BlockSpec((bn_blk, D), lambda i: (i, 0)),
            pl.BlockSpec((NC, bn_blk, D), lambda i: (0, i, 0)),
            pl.BlockSpec((NW, bn_blk), lambda i: (0, i)),
            pl.BlockSpec((D, D), lambda i: (0, 0)),
            pl.BlockSpec((D, D), lambda i: (0, 0)),
            pl.BlockSpec((1, D), lambda i: (0, 0)),
            pl.BlockSpec((1, D), lambda i: (0, 0)),
            pl.BlockSpec((1, D), lambda i: (0, 0)),
            pl.BlockSpec((D, D), lambda i: (0, 0)),
            pl.BlockSpec((1, D), lambda i: (0, 0)),
            pl.BlockSpec((1, D), lambda i: (0, 0)),
            pl.BlockSpec((1, D), lambda i: (0, 0)),
        ],
        out_specs=pl.BlockSpec((bn_blk, D), lambda i: (i, 0)),
        out_shape=jax.ShapeDtypeStruct((NPAD, D), jnp.float32),
    )(x, parts, cnth, w1x, w1a, b1, g1, be1, w2, b2, gn, bn)


def kernel(x, coords, edge_index, edge_attr,
           W1e, b1e, g1e, be1e, W2e, b2e, Wse, bse,
           W1n, b1n, g1n, be1n, W2n, b2n, gnn, bnn):
    src = edge_index[0].astype(jnp.int32)
    dst = edge_index[1].astype(jnp.int32)
    w_src = W1e[:D]
    w_dst = W1e[D:2 * D]
    w_ea = W1e[2 * D:2 * D + A]
    w_rbf = W1e[2 * D + A:]

    tp, tq = _prep(x, w_src, w_dst, b1e.reshape(1, D))
    summ, rd = _make_sc_gather()(tp, tq, coords[:, 0], coords[:, 1],
                                 coords[:, 2], src, dst)
    m = _edge(summ, rd.reshape(E, 1), edge_attr,
              w_ea, w_rbf, g1e.reshape(1, D), be1e.reshape(1, D),
              W2e, b2e.reshape(1, D), Wse, bse.reshape(1, 1))
    parts, cnth = _make_sc_scatter()(m, dst)
    x_pad = jnp.pad(x, ((0, NPAD - N), (0, 0)))
    out = _node(x_pad, parts, cnth, W1n[:D], W1n[D:], b1n.reshape(1, D),
                g1n.reshape(1, D), be1n.reshape(1, D),
                W2n, b2n.reshape(1, D), gnn.reshape(1, D), bnn.reshape(1, D))
    return out[:N]


# speedup vs baseline: 6.0132x; 1.0384x over previous
import functools

import jax
import jax.numpy as jnp
from jax import lax
from jax.experimental import pallas as pl
from jax.experimental.pallas import tpu as pltpu
from jax.experimental.pallas import tpu_sc as plsc

N = 10000
E = 320000
D = 128
A = 16
MW = 144
NPAD = 10240

NC = 2
NS = 16
NW = NC * NS
EPW = E // NW
C = 80
NCHUNK = EPW // C

_SIGMAS = (0.1, 0.5, 1.0)


def _prep_body(x_ref, ws_ref, wd_ref, b_ref, tp_ref, tq_ref):
    xb = x_ref[...]
    tp_ref[...] = (
        jnp.dot(xb, ws_ref[...], preferred_element_type=jnp.float32) + b_ref[...]
    )
    tq_ref[...] = jnp.dot(xb, wd_ref[...], preferred_element_type=jnp.float32)


def _prep(x, w_src, w_dst, b1e):
    bn = 1000
    return pl.pallas_call(
        _prep_body,
        grid=(N // bn,),
        in_specs=[
            pl.BlockSpec((bn, D), lambda i: (i, 0)),
            pl.BlockSpec((D, D), lambda i: (0, 0)),
            pl.BlockSpec((D, D), lambda i: (0, 0)),
            pl.BlockSpec((1, D), lambda i: (0, 0)),
        ],
        out_specs=[
            pl.BlockSpec((bn, D), lambda i: (i, 0)),
            pl.BlockSpec((bn, D), lambda i: (i, 0)),
        ],
        out_shape=[
            jax.ShapeDtypeStruct((N, D), jnp.float32),
            jax.ShapeDtypeStruct((N, D), jnp.float32),
        ],
    )(x, w_src, w_dst, b1e)


@functools.lru_cache(maxsize=None)
def _make_mesh():
    return plsc.VectorSubcoreMesh(
        core_axis_name="c", subcore_axis_name="s",
        num_cores=NC, num_subcores=NS)


@functools.lru_cache(maxsize=None)
def _make_sc_gather():
    return functools.partial(
        pl.kernel,
        out_type=(
            jax.ShapeDtypeStruct((E, D), jnp.float32),
            jax.ShapeDtypeStruct((E,), jnp.float32),
        ),
        mesh=_make_mesh(),
        compiler_params=pltpu.CompilerParams(needs_layout_passes=False),
        scratch_types=[
            pltpu.VMEM((N,), jnp.float32),
            pltpu.VMEM((N,), jnp.float32),
            pltpu.VMEM((N,), jnp.float32),
            pltpu.VMEM((C,), jnp.int32),
            pltpu.VMEM((C,), jnp.int32),
            pltpu.VMEM((C, D), jnp.float32),
            pltpu.VMEM((C, D), jnp.float32),
            pltpu.VMEM((C,), jnp.float32),
            pltpu.VMEM((C,), jnp.int32),
            pltpu.VMEM((C,), jnp.int32),
            pltpu.VMEM((C, D), jnp.float32),
            pltpu.VMEM((C, D), jnp.float32),
            pltpu.VMEM((C,), jnp.float32),
            pltpu.SemaphoreType.DMA,
            pltpu.SemaphoreType.DMA,
        ],
    )(_sc_gather_body)


def _sc_gather_body(tp_hbm, tq_hbm, cx_hbm, cy_hbm, cz_hbm, src_hbm, dst_hbm,
                    sum_out, rd_out,
                    cx_v, cy_v, cz_v,
                    src0, dst0, bufp0, bufq0, rd0,
                    src1, dst1, bufp1, bufq1, rd1,
                    sem0, sem1):
    wid = lax.axis_index("s") * NC + lax.axis_index("c")
    pltpu.sync_copy(cx_hbm, cx_v)
    pltpu.sync_copy(cy_hbm, cy_v)
    pltpu.sync_copy(cz_hbm, cz_v)
    sets = ((src0, dst0, bufp0, bufq0, rd0, sem0),
            (src1, dst1, bufp1, bufq1, rd1, sem1))

    def stage_in(c, s):
        src_v, dst_v, bufp, bufq, _, sem = sets[s]
        base = wid * EPW + c * C
        pltpu.sync_copy(src_hbm.at[pl.ds(base, C)], src_v)
        pltpu.sync_copy(dst_hbm.at[pl.ds(base, C)], dst_v)
        pltpu.async_copy(tp_hbm.at[src_v], bufp, sem)
        pltpu.async_copy(tq_hbm.at[dst_v], bufq, sem)

    def process(c, s):
        src_v, dst_v, bufp, bufq, rd_v, sem = sets[s]
        base = wid * EPW + c * C
        for g in range(C // 16):
            s16 = src_v[pl.ds(g * 16, 16)]
            d16 = dst_v[pl.ds(g * 16, 16)]
            acc = jnp.zeros((16,), jnp.float32)
            for cv in (cx_v, cy_v, cz_v):
                r = plsc.load_gather(cv, [s16]) - plsc.load_gather(cv, [d16])
                acc = acc + r * r
            rd_v[pl.ds(g * 16, 16)] = acc
        pltpu.make_async_copy(tp_hbm.at[src_v], bufp, sem).wait()
        pltpu.make_async_copy(tq_hbm.at[dst_v], bufq, sem).wait()

        def row(i, _):
            for k in range(D // 16):
                sl = pl.ds(k * 16, 16)
                plsc.addupdate(bufp.at[i, sl], bufq[i, sl])
            return 0

        lax.fori_loop(0, C, row, 0, unroll=False)
        pltpu.sync_copy(bufp, sum_out.at[pl.ds(base, C)])
        pltpu.sync_copy(rd_v, rd_out.at[pl.ds(base, C)])

    stage_in(0, 0)

    def group(g, _):
        c0 = 2 * g
        stage_in(c0 + 1, 1)
        process(c0, 0)
        stage_in(c0 + 2, 0)
        process(c0 + 1, 1)
        return 0

    lax.fori_loop(0, (NCHUNK - 1) // 2, group, 0, unroll=False)
    process(NCHUNK - 1, 0)


def _edge_body(sum_ref, rd_ref, ea_ref, wea_ref, wrbf_ref, g1_ref, be1_ref,
               w2_ref, b2_ref, wse_ref, bse_ref, out_ref):
    b2 = sum_ref.shape[0]
    h1 = sum_ref[...] + jnp.dot(
        ea_ref[...], wea_ref[...], preferred_element_type=jnp.float32)
    d2 = rd_ref[...] + 1e-8
    wr = wrbf_ref[...]
    for k, sig in enumerate(_SIGMAS):
        h1 = h1 + jnp.exp(d2 * (-1.0 / (2.0 * sig * sig))) * wr[k:k + 1, :]
    h = h1 * jax.nn.sigmoid(h1)
    mu = jnp.mean(h, axis=-1, keepdims=True)
    var = jnp.mean((h - mu) ** 2, axis=-1, keepdims=True)
    h = (h - mu) * lax.rsqrt(var + 1e-5) * g1_ref[...] + be1_ref[...]
    m = jnp.dot(h.astype(jnp.bfloat16), w2_ref[...].astype(jnp.bfloat16),
                preferred_element_type=jnp.float32) + b2_ref[...]
    w = jax.nn.sigmoid(
        jnp.dot(m, wse_ref[...], preferred_element_type=jnp.float32) + bse_ref[...])
    m = m * w
    out_ref[...] = m


def _edge(summ, rd, ea, wea, wrbf, g1, be1, w2, b2, wse, bse):
    blk = 4000
    return pl.pallas_call(
        _edge_body,
        grid=(E // blk,),
        in_specs=[
            pl.BlockSpec((blk, D), lambda i: (i, 0)),
            pl.BlockSpec((blk, 1), lambda i: (i, 0)),
            pl.BlockSpec((blk, A), lambda i: (i, 0)),
            pl.BlockSpec((A, D), lambda i: (0, 0)),
            pl.BlockSpec((3, D), lambda i: (0, 0)),
            pl.BlockSpec((1, D), lambda i: (0, 0)),
            pl.BlockSpec((1, D), lambda i: (0, 0)),
            pl.BlockSpec((D, D), lambda i: (0, 0)),
            pl.BlockSpec((1, D), lambda i: (0, 0)),
            pl.BlockSpec((D, 1), lambda i: (0, 0)),
            pl.BlockSpec((1, 1), lambda i: (0, 0)),
        ],
        out_specs=pl.BlockSpec((blk, D), lambda i: (i, 0)),
        out_shape=jax.ShapeDtypeStruct((E, D), jnp.float32),
    )(summ, rd, ea, wea, wrbf, g1, be1, w2, b2, wse, bse)


@functools.lru_cache(maxsize=None)
def _make_sc_scatter():
    return functools.partial(
        pl.kernel,
        out_type=(
            jax.ShapeDtypeStruct((NC, NPAD, D), jnp.float32),
            jax.ShapeDtypeStruct((NW, NPAD), jnp.int32),
        ),
        mesh=_make_mesh(),
        compiler_params=pltpu.CompilerParams(needs_layout_passes=False),
        scratch_types=[
            pltpu.VMEM((C, D), jnp.float32),
            pltpu.VMEM((C,), jnp.int32),
            pltpu.VMEM((C, D), jnp.float32),
            pltpu.VMEM((C,), jnp.int32),
            pltpu.VMEM((NPAD,), jnp.int32),
            pltpu.VMEM_SHARED((NPAD, D), jnp.float32),
            pltpu.SemaphoreType.DMA,
            pltpu.SemaphoreType.DMA,
        ],
    )(_sc_scatter_body)


def _sc_scatter_body(m_hbm, dst_hbm, out_hbm, cnt_hbm,
                     m0, di0, m1, di1, cnt_v, acc_sh, sem0, sem1):
    cid = lax.axis_index("c")
    sid = lax.axis_index("s")
    wid = sid * NC + cid
    rows_per_tile = NPAD // NS
    sets = ((m0, di0, sem0), (m1, di1, sem1))

    def zrow(i, _):
        for k in range(D // 16):
            m0[i, pl.ds(k * 16, 16)] = jnp.zeros((16,), jnp.float32)
        return 0

    lax.fori_loop(0, C, zrow, 0, unroll=False)

    def zcnt(i, _):
        cnt_v[pl.ds(i * 16, 16)] = jnp.zeros((16,), jnp.int32)
        return 0

    lax.fori_loop(0, NPAD // 16, zcnt, 0, unroll=False)

    def zcp(b, _):
        pltpu.sync_copy(m0, acc_sh.at[pl.ds(sid * rows_per_tile + b * C, C)])
        return 0

    lax.fori_loop(0, rows_per_tile // C, zcp, 0, unroll=False)
    plsc.subcore_barrier()

    def stage_in(c, s):
        m_v, dsti_v, sem = sets[s]
        base = wid * EPW + c * C
        pltpu.sync_copy(dst_hbm.at[pl.ds(base, C)], dsti_v)
        pltpu.async_copy(m_hbm.at[pl.ds(base, C)], m_v, sem)

    def process(c, s):
        m_v, dsti_v, sem = sets[s]
        base = wid * EPW + c * C
        for g in range(C // 16):
            v16 = dsti_v[pl.ds(g * 16, 16)]
            cnts, last = plsc.scan_count(v16)
            cur = plsc.load_gather(cnt_v, [v16])
            plsc.store_scatter(cnt_v, [v16], cur + cnts, mask=last)
        pltpu.make_async_copy(m_hbm.at[pl.ds(base, C)], m_v, sem).wait()
        pltpu.sync_copy(m_v, acc_sh.at[dsti_v], add=True)

    stage_in(0, 0)

    def group(g, _):
        c0 = 2 * g
        stage_in(c0 + 1, 1)
        process(c0, 0)
        stage_in(c0 + 2, 0)
        process(c0 + 1, 1)
        return 0

    lax.fori_loop(0, (NCHUNK - 1) // 2, group, 0, unroll=False)
    process(NCHUNK - 1, 0)
    pltpu.sync_copy(cnt_v, cnt_hbm.at[wid])
    plsc.subcore_barrier()

    def ocp(b, _):
        r0 = sid * rows_per_tile + b * C
        pltpu.sync_copy(acc_sh.at[pl.ds(r0, C)], m0)
        pltpu.sync_copy(m0, out_hbm.at[cid].at[pl.ds(r0, C)])
        return 0

    lax.fori_loop(0, rows_per_tile // C, ocp, 0, unroll=False)


def _node_body(x_ref, parts_ref, cnt_ref, w1x_ref, w1a_ref, b1_ref, g1_ref,
               be1_ref, w2_ref, b2_ref, gn_ref, bn_ref, out_ref):
    xb = x_ref[...]
    p = parts_ref[0] + parts_ref[1]
    cnt = jnp.sum(cnt_ref[...], axis=0).astype(jnp.float32)[:, None]
    agg = p / jnp.maximum(cnt, 1.0)
    h = (jnp.dot(xb, w1x_ref[...], preferred_element_type=jnp.float32)
         + jnp.dot(agg, w1a_ref[...], preferred_element_type=jnp.float32)
         + b1_ref[...])
    h = h * jax.nn.sigmoid(h)
    mu = jnp.mean(h, axis=-1, keepdims=True)
    var = jnp.mean((h - mu) ** 2, axis=-1, keepdims=True)
    h = (h - mu) * lax.rsqrt(var + 1e-5) * g1_ref[...] + be1_ref[...]
    h = jnp.dot(h, w2_ref[...], preferred_element_type=jnp.float32) + b2_ref[...]
    mu = jnp.mean(h, axis=-1, keepdims=True)
    var = jnp.mean((h - mu) ** 2, axis=-1, keepdims=True)
    h = (h - mu) * lax.rsqrt(var + 1e-5) * gn_ref[...] + bn_ref[...]
    out_ref[...] = xb + h


def _node(x, parts, cnth, w1x, w1a, b1, g1, be1, w2, b2, gn, bn):
    bn_blk = 1024
    return pl.pallas_call(
        _node_body,
        grid=(NPAD // bn_blk,),
        in_specs=[
            pl.BlockSpec((bn_blk, D), lambda i: (i, 0)),
            pl.BlockSpec((NC, bn_blk, D), lambda i: (0, i, 0)),
            pl.BlockSpec((NW, bn_blk), lambda i: (0, i)),
            pl.BlockSpec((D, D), lambda i: (0, 0)),
            pl.BlockSpec((D, D), lambda i: (0, 0)),
            pl.BlockSpec((1, D), lambda i: (0, 0)),
            pl.BlockSpec((1, D), lambda i: (0, 0)),
            pl.BlockSpec((1, D), lambda i: (0, 0)),
            pl.BlockSpec((D, D), lambda i: (0, 0)),
            pl.BlockSpec((1, D), lambda i: (0, 0)),
            pl.BlockSpec((1, D), lambda i: (0, 0)),
            pl.BlockSpec((1, D), lambda i: (0, 0)),
        ],
        out_specs=pl.BlockSpec((bn_blk, D), lambda i: (i, 0)),
        out_shape=jax.ShapeDtypeStruct((NPAD, D), jnp.float32),
    )(x, parts, cnth, w1x, w1a, b1, g1, be1, w2, b2, gn, bn)


def kernel(x, coords, edge_index, edge_attr,
           W1e, b1e, g1e, be1e, W2e, b2e, Wse, bse,
           W1n, b1n, g1n, be1n, W2n, b2n, gnn, bnn):
    src = edge_index[0].astype(jnp.int32)
    dst = edge_index[1].astype(jnp.int32)
    w_src = W1e[:D]
    w_dst = W1e[D:2 * D]
    w_ea = W1e[2 * D:2 * D + A]
    w_rbf = W1e[2 * D + A:]

    tp, tq = _prep(x, w_src, w_dst, b1e.reshape(1, D))
    summ, rd = _make_sc_gather()(tp, tq, coords[:, 0], coords[:, 1],
                                 coords[:, 2], src, dst)
    m = _edge(summ, rd.reshape(E, 1), edge_attr,
              w_ea, w_rbf, g1e.reshape(1, D), be1e.reshape(1, D),
              W2e, b2e.reshape(1, D), Wse, bse.reshape(1, 1))
    parts, cnth = _make_sc_scatter()(m, dst)
    x_pad = jnp.pad(x, ((0, NPAD - N), (0, 0)))
    out = _node(x_pad, parts, cnth, W1n[:D], W1n[D:], b1n.reshape(1, D),
                g1n.reshape(1, D), be1n.reshape(1, D),
                W2n, b2n.reshape(1, D), gnn.reshape(1, D), bnn.reshape(1, D))
    return out[:N]


# speedup vs baseline: 6.2489x; 1.0392x over previous
import functools

import jax
import jax.numpy as jnp
from jax import lax
from jax.experimental import pallas as pl
from jax.experimental.pallas import tpu as pltpu
from jax.experimental.pallas import tpu_sc as plsc

N = 10000
E = 320000
D = 128
A = 16
MW = 144
NPAD = 10240

NC = 2
NS = 16
NW = NC * NS
NSLICE = 2
ES = E // NSLICE
EPW = ES // NW
C = 40
NCHUNK = EPW // C

_SIGMAS = (0.1, 0.5, 1.0)


def _prep_body(x_ref, ws_ref, wd_ref, b_ref, tp_ref, tq_ref):
    xb = x_ref[...]
    tp_ref[...] = (
        jnp.dot(xb, ws_ref[...], preferred_element_type=jnp.float32) + b_ref[...]
    )
    tq_ref[...] = jnp.dot(xb, wd_ref[...], preferred_element_type=jnp.float32)


def _prep(x, w_src, w_dst, b1e):
    bn = 1000
    return pl.pallas_call(
        _prep_body,
        grid=(N // bn,),
        in_specs=[
            pl.BlockSpec((bn, D), lambda i: (i, 0)),
            pl.BlockSpec((D, D), lambda i: (0, 0)),
            pl.BlockSpec((D, D), lambda i: (0, 0)),
            pl.BlockSpec((1, D), lambda i: (0, 0)),
        ],
        out_specs=[
            pl.BlockSpec((bn, D), lambda i: (i, 0)),
            pl.BlockSpec((bn, D), lambda i: (i, 0)),
        ],
        out_shape=[
            jax.ShapeDtypeStruct((N, D), jnp.float32),
            jax.ShapeDtypeStruct((N, D), jnp.float32),
        ],
    )(x, w_src, w_dst, b1e)


@functools.lru_cache(maxsize=None)
def _make_mesh():
    return plsc.VectorSubcoreMesh(
        core_axis_name="c", subcore_axis_name="s",
        num_cores=NC, num_subcores=NS)


@functools.lru_cache(maxsize=None)
def _make_sc_gather():
    return functools.partial(
        pl.kernel,
        out_type=(
            jax.ShapeDtypeStruct((ES, D), jnp.float32),
            jax.ShapeDtypeStruct((ES,), jnp.float32),
        ),
        mesh=_make_mesh(),
        compiler_params=pltpu.CompilerParams(needs_layout_passes=False),
        scratch_types=[
            pltpu.VMEM((N,), jnp.float32),
            pltpu.VMEM((N,), jnp.float32),
            pltpu.VMEM((N,), jnp.float32),
            pltpu.VMEM((C,), jnp.int32),
            pltpu.VMEM((C,), jnp.int32),
            pltpu.VMEM((C, D), jnp.float32),
            pltpu.VMEM((C, D), jnp.float32),
            pltpu.VMEM((C,), jnp.float32),
            pltpu.VMEM((C,), jnp.int32),
            pltpu.VMEM((C,), jnp.int32),
            pltpu.VMEM((C, D), jnp.float32),
            pltpu.VMEM((C, D), jnp.float32),
            pltpu.VMEM((C,), jnp.float32),
            pltpu.SemaphoreType.DMA,
            pltpu.SemaphoreType.DMA,
        ],
    )(_sc_gather_body)


def _sc_gather_body(tp_hbm, tq_hbm, cx_hbm, cy_hbm, cz_hbm, src_hbm, dst_hbm,
                    sum_out, rd_out,
                    cx_v, cy_v, cz_v,
                    src0, dst0, bufp0, bufq0, rd0,
                    src1, dst1, bufp1, bufq1, rd1,
                    sem0, sem1):
    wid = lax.axis_index("s") * NC + lax.axis_index("c")
    pltpu.sync_copy(cx_hbm, cx_v)
    pltpu.sync_copy(cy_hbm, cy_v)
    pltpu.sync_copy(cz_hbm, cz_v)
    sets = ((src0, dst0, bufp0, bufq0, rd0, sem0),
            (src1, dst1, bufp1, bufq1, rd1, sem1))

    def stage_in(c, s):
        src_v, dst_v, bufp, bufq, _, sem = sets[s]
        base = wid * EPW + c * C
        pltpu.sync_copy(src_hbm.at[pl.ds(base, C)], src_v)
        pltpu.sync_copy(dst_hbm.at[pl.ds(base, C)], dst_v)
        pltpu.async_copy(tp_hbm.at[src_v], bufp, sem)
        pltpu.async_copy(tq_hbm.at[dst_v], bufq, sem)

    def process(c, s):
        src_v, dst_v, bufp, bufq, rd_v, sem = sets[s]
        base = wid * EPW + c * C
        for g in range(C // 16):
            s16 = src_v[pl.ds(g * 16, 16)]
            d16 = dst_v[pl.ds(g * 16, 16)]
            acc = jnp.zeros((16,), jnp.float32)
            for cv in (cx_v, cy_v, cz_v):
                r = plsc.load_gather(cv, [s16]) - plsc.load_gather(cv, [d16])
                acc = acc + r * r
            rd_v[pl.ds(g * 16, 16)] = acc
        pltpu.make_async_copy(tp_hbm.at[src_v], bufp, sem).wait()
        pltpu.make_async_copy(tq_hbm.at[dst_v], bufq, sem).wait()

        def row(i, _):
            for k in range(D // 16):
                sl = pl.ds(k * 16, 16)
                plsc.addupdate(bufp.at[i, sl], bufq[i, sl])
            return 0

        lax.fori_loop(0, C, row, 0, unroll=False)
        pltpu.sync_copy(bufp, sum_out.at[pl.ds(base, C)])
        pltpu.sync_copy(rd_v, rd_out.at[pl.ds(base, C)])

    stage_in(0, 0)

    def group(g, _):
        c0 = 2 * g
        stage_in(c0 + 1, 1)
        process(c0, 0)
        stage_in(c0 + 2, 0)
        process(c0 + 1, 1)
        return 0

    lax.fori_loop(0, (NCHUNK - 1) // 2, group, 0, unroll=False)
    process(NCHUNK - 1, 0)


def _edge_body(sum_ref, rd_ref, ea_ref, wea_ref, wrbf_ref, g1_ref, be1_ref,
               w2_ref, b2_ref, wse_ref, bse_ref, out_ref):
    b2 = sum_ref.shape[0]
    h1 = sum_ref[...].astype(jnp.float32) + jnp.dot(
        ea_ref[...], wea_ref[...], preferred_element_type=jnp.float32)
    d2 = rd_ref[...] + 1e-8
    wr = wrbf_ref[...]
    for k, sig in enumerate(_SIGMAS):
        h1 = h1 + jnp.exp(d2 * (-1.0 / (2.0 * sig * sig))) * wr[k:k + 1, :]
    h = h1 * jax.nn.sigmoid(h1)
    mu = jnp.mean(h, axis=-1, keepdims=True)
    var = jnp.mean((h - mu) ** 2, axis=-1, keepdims=True)
    h = (h - mu) * lax.rsqrt(var + 1e-5) * g1_ref[...] + be1_ref[...]
    m = jnp.dot(h.astype(jnp.bfloat16), w2_ref[...].astype(jnp.bfloat16),
                preferred_element_type=jnp.float32) + b2_ref[...]
    w = jax.nn.sigmoid(
        jnp.dot(m, wse_ref[...], preferred_element_type=jnp.float32) + bse_ref[...])
    m = m * w
    out_ref[...] = m


def _edge(summ, rd, ea, wea, wrbf, g1, be1, w2, b2, wse, bse):
    blk = 4000
    return pl.pallas_call(
        _edge_body,
        grid=(ES // blk,),
        in_specs=[
            pl.BlockSpec((blk, D), lambda i: (i, 0)),
            pl.BlockSpec((blk, 1), lambda i: (i, 0)),
            pl.BlockSpec((blk, A), lambda i: (i, 0)),
            pl.BlockSpec((A, D), lambda i: (0, 0)),
            pl.BlockSpec((3, D), lambda i: (0, 0)),
            pl.BlockSpec((1, D), lambda i: (0, 0)),
            pl.BlockSpec((1, D), lambda i: (0, 0)),
            pl.BlockSpec((D, D), lambda i: (0, 0)),
            pl.BlockSpec((1, D), lambda i: (0, 0)),
            pl.BlockSpec((D, 1), lambda i: (0, 0)),
            pl.BlockSpec((1, 1), lambda i: (0, 0)),
        ],
        out_specs=pl.BlockSpec((blk, D), lambda i: (i, 0)),
        out_shape=jax.ShapeDtypeStruct((ES, D), jnp.float32),
    )(summ, rd, ea, wea, wrbf, g1, be1, w2, b2, wse, bse)


@functools.lru_cache(maxsize=None)
def _make_sc_scatter():
    return functools.partial(
        pl.kernel,
        out_type=(
            jax.ShapeDtypeStruct((NC, NPAD, D), jnp.float32),
            jax.ShapeDtypeStruct((NW, NPAD), jnp.int32),
        ),
        mesh=_make_mesh(),
        compiler_params=pltpu.CompilerParams(needs_layout_passes=False),
        scratch_types=[
            pltpu.VMEM((C, D), jnp.float32),
            pltpu.VMEM((C,), jnp.int32),
            pltpu.VMEM((C, D), jnp.float32),
            pltpu.VMEM((C,), jnp.int32),
            pltpu.VMEM((NPAD,), jnp.int32),
            pltpu.VMEM_SHARED((NPAD, D), jnp.float32),
            pltpu.SemaphoreType.DMA,
            pltpu.SemaphoreType.DMA,
        ],
    )(_sc_scatter_body)


def _sc_scatter_body(m_hbm, dst_hbm, out_hbm, cnt_hbm,
                     m0, di0, m1, di1, cnt_v, acc_sh, sem0, sem1):
    cid = lax.axis_index("c")
    sid = lax.axis_index("s")
    wid = sid * NC + cid
    rows_per_tile = NPAD // NS
    sets = ((m0, di0, sem0), (m1, di1, sem1))

    def zrow(i, _):
        for k in range(D // 16):
            m0[i, pl.ds(k * 16, 16)] = jnp.zeros((16,), jnp.float32)
        return 0

    lax.fori_loop(0, C, zrow, 0, unroll=False)

    def zcnt(i, _):
        cnt_v[pl.ds(i * 16, 16)] = jnp.zeros((16,), jnp.int32)
        return 0

    lax.fori_loop(0, NPAD // 16, zcnt, 0, unroll=False)

    def zcp(b, _):
        pltpu.sync_copy(m0, acc_sh.at[pl.ds(sid * rows_per_tile + b * C, C)])
        return 0

    lax.fori_loop(0, rows_per_tile // C, zcp, 0, unroll=False)
    plsc.subcore_barrier()

    def stage_in(c, s):
        m_v, dsti_v, sem = sets[s]
        base = wid * EPW + c * C
        pltpu.sync_copy(dst_hbm.at[pl.ds(base, C)], dsti_v)
        pltpu.async_copy(m_hbm.at[pl.ds(base, C)], m_v, sem)

    def process(c, s):
        m_v, dsti_v, sem = sets[s]
        base = wid * EPW + c * C
        for g in range(C // 16):
            v16 = dsti_v[pl.ds(g * 16, 16)]
            cnts, last = plsc.scan_count(v16)
            cur = plsc.load_gather(cnt_v, [v16])
            plsc.store_scatter(cnt_v, [v16], cur + cnts, mask=last)
        pltpu.make_async_copy(m_hbm.at[pl.ds(base, C)], m_v, sem).wait()
        pltpu.sync_copy(m_v, acc_sh.at[dsti_v], add=True)

    stage_in(0, 0)

    def group(g, _):
        c0 = 2 * g
        stage_in(c0 + 1, 1)
        process(c0, 0)
        stage_in(c0 + 2, 0)
        process(c0 + 1, 1)
        return 0

    lax.fori_loop(0, (NCHUNK - 1) // 2, group, 0, unroll=False)
    process(NCHUNK - 1, 0)
    pltpu.sync_copy(cnt_v, cnt_hbm.at[wid])
    plsc.subcore_barrier()

    def ocp(b, _):
        r0 = sid * rows_per_tile + b * C
        pltpu.sync_copy(acc_sh.at[pl.ds(r0, C)], m0)
        pltpu.sync_copy(m0, out_hbm.at[cid].at[pl.ds(r0, C)])
        return 0

    lax.fori_loop(0, rows_per_tile // C, ocp, 0, unroll=False)


def _node_body(x_ref, parts0_ref, parts1_ref, cnt0_ref, cnt1_ref, w1x_ref,
               w1a_ref, b1_ref, g1_ref, be1_ref, w2_ref, b2_ref, gn_ref,
               bn_ref, out_ref):
    xb = x_ref[...]
    p = (parts0_ref[0] + parts0_ref[1]) + (parts1_ref[0] + parts1_ref[1])
    cnt = (jnp.sum(cnt0_ref[...], axis=0)
           + jnp.sum(cnt1_ref[...], axis=0)).astype(jnp.float32)[:, None]
    agg = p / jnp.maximum(cnt, 1.0)
    h = (jnp.dot(xb, w1x_ref[...], preferred_element_type=jnp.float32)
         + jnp.dot(agg, w1a_ref[...], preferred_element_type=jnp.float32)
         + b1_ref[...])
    h = h * jax.nn.sigmoid(h)
    mu = jnp.mean(h, axis=-1, keepdims=True)
    var = jnp.mean((h - mu) ** 2, axis=-1, keepdims=True)
    h = (h - mu) * lax.rsqrt(var + 1e-5) * g1_ref[...] + be1_ref[...]
    h = jnp.dot(h, w2_ref[...], preferred_element_type=jnp.float32) + b2_ref[...]
    mu = jnp.mean(h, axis=-1, keepdims=True)
    var = jnp.mean((h - mu) ** 2, axis=-1, keepdims=True)
    h = (h - mu) * lax.rsqrt(var + 1e-5) * gn_ref[...] + bn_ref[...]
    out_ref[...] = xb + h


def _node(x, parts0, parts1, cnth0, cnth1, w1x, w1a, b1, g1, be1, w2, b2,
          gn, bn):
    bn_blk = 1024
    return pl.pallas_call(
        _node_body,
        grid=(NPAD // bn_blk,),
        in_specs=[
            pl.BlockSpec((bn_blk, D), lambda i: (i, 0)),
            pl.BlockSpec((NC, bn_blk, D), lambda i: (0, i, 0)),
            pl.BlockSpec((NC, bn_blk, D), lambda i: (0, i, 0)),
            pl.BlockSpec((NW, bn_blk), lambda i: (0, i)),
            pl.BlockSpec((NW, bn_blk), lambda i: (0, i)),
            pl.BlockSpec((D, D), lambda i: (0, 0)),
            pl.BlockSpec((D, D), lambda i: (0, 0)),
            pl.BlockSpec((1, D), lambda i: (0, 0)),
            pl.BlockSpec((1, D), lambda i: (0, 0)),
            pl.BlockSpec((1, D), lambda i: (0, 0)),
            pl.BlockSpec((D, D), lambda i: (0, 0)),
            pl.BlockSpec((1, D), lambda i: (0, 0)),
            pl.BlockSpec((1, D), lambda i: (0, 0)),
            pl.BlockSpec((1, D), lambda i: (0, 0)),
        ],
        out_specs=pl.BlockSpec((bn_blk, D), lambda i: (i, 0)),
        out_shape=jax.ShapeDtypeStruct((NPAD, D), jnp.float32),
    )(x, parts0, parts1, cnth0, cnth1, w1x, w1a, b1, g1, be1, w2, b2, gn, bn)


def kernel(x, coords, edge_index, edge_attr,
           W1e, b1e, g1e, be1e, W2e, b2e, Wse, bse,
           W1n, b1n, g1n, be1n, W2n, b2n, gnn, bnn):
    src = edge_index[0].astype(jnp.int32)
    dst = edge_index[1].astype(jnp.int32)
    w_src = W1e[:D]
    w_dst = W1e[D:2 * D]
    w_ea = W1e[2 * D:2 * D + A]
    w_rbf = W1e[2 * D + A:]

    tp, tq = _prep(x, w_src, w_dst, b1e.reshape(1, D))
    cx, cy, cz = coords[:, 0], coords[:, 1], coords[:, 2]
    gather = _make_sc_gather()
    scatter = _make_sc_scatter()
    parts, cnts = [], []
    ms, dsts = [], []
    for s in range(NSLICE):
        sl = slice(s * ES, (s + 1) * ES)
        summ, rd = gather(tp, tq, cx, cy, cz, src[sl], dst[sl])
        m = _edge(summ, rd.reshape(ES, 1), edge_attr[sl],
                  w_ea, w_rbf, g1e.reshape(1, D), be1e.reshape(1, D),
                  W2e, b2e.reshape(1, D), Wse, bse.reshape(1, 1))
        ms.append(m)
        dsts.append(dst[sl])
    for s in range(NSLICE):
        p, c = scatter(ms[s], dsts[s])
        parts.append(p)
        cnts.append(c)
    x_pad = jnp.pad(x, ((0, NPAD - N), (0, 0)))
    out = _node(x_pad, parts[0], parts[1], cnts[0], cnts[1],
                W1n[:D], W1n[D:], b1n.reshape(1, D),
                g1n.reshape(1, D), be1n.reshape(1, D),
                W2n, b2n.reshape(1, D), gnn.reshape(1, D), bnn.reshape(1, D))
    return out[:N]
